# baseline (speedup 1.0000x reference)
import functools

import jax
import jax.numpy as jnp
from jax import lax
from jax.experimental import pallas as pl
from jax.experimental.pallas import tpu as pltpu
from jax.experimental.pallas import tpu_sc as plsc

_CHUNK = 8


def _sc_gather(t, ac, om):
    (B,) = t.shape
    info = plsc.get_sparse_core_info()
    nc = info.num_cores
    n_active = B // _CHUNK
    mesh = plsc.VectorSubcoreMesh(core_axis_name="c", subcore_axis_name="s")

    @functools.partial(
        pl.kernel,
        mesh=mesh,
        out_type=[
            jax.ShapeDtypeStruct((B,), jnp.float32),
            jax.ShapeDtypeStruct((B,), jnp.float32),
        ],
        scratch_types=[
            pltpu.VMEM((_CHUNK,), jnp.int32),
            pltpu.VMEM((_CHUNK,), jnp.float32),
            pltpu.SemaphoreType.DMA,
        ],
    )
    def gather_kernel(t_hbm, ac_hbm, om_hbm, c1_hbm, c2_hbm, idx_v, row_v, sem):
        wid = lax.axis_index("s") * nc + lax.axis_index("c")

        @pl.when(wid < n_active)
        def _():
            base = wid * _CHUNK
            pltpu.sync_copy(t_hbm.at[pl.ds(base, _CHUNK)], idx_v)
            pltpu.async_copy(ac_hbm.at[idx_v], row_v, sem).wait()
            pltpu.sync_copy(row_v, c1_hbm.at[pl.ds(base, _CHUNK)])
            pltpu.async_copy(om_hbm.at[idx_v], row_v, sem).wait()
            pltpu.sync_copy(row_v, c2_hbm.at[pl.ds(base, _CHUNK)])

    return gather_kernel(t, ac, om)


def _combine_body(c1_ref, c2_ref, x_ref, n_ref, o_ref):
    b = pl.program_id(0)
    o_ref[...] = c1_ref[b] * x_ref[...] + c2_ref[b] * n_ref[...]


def kernel(x_start, t, noise, sqrt_alphas_cumprod, sqrt_one_minus_alphas_cumprod):
    B, C, H, W = x_start.shape

    c1, c2 = _sc_gather(t.astype(jnp.int32), sqrt_alphas_cumprod,
                        sqrt_one_minus_alphas_cumprod)

    smem = pl.BlockSpec(memory_space=pltpu.SMEM)
    blk = pl.BlockSpec((1, C, H, W), lambda b: (b, 0, 0, 0))

    out = pl.pallas_call(
        _combine_body,
        grid=(B,),
        in_specs=[smem, smem, blk, blk],
        out_specs=blk,
        out_shape=jax.ShapeDtypeStruct((B, C, H, W), jnp.float32),
    )(c1, c2, x_start, noise)
    return out
